# Initial kernel scaffold; baseline (speedup 1.0000x reference)
import jax
import jax.numpy as jnp
from jax.experimental import pallas as pl


def kernel(x, edge_index, batch, W1_0, W2_0, mg0, mb0, og0, ob0, W1_1, W2_1, mg1, mb1, og1, ob1):
    raise NotImplementedError("write your pallas kernel here")


# speedup vs baseline: 7.7577x; 7.7577x over previous
import functools

import jax
import jax.numpy as jnp
from jax import lax
from jax.experimental import pallas as pl
from jax.experimental.pallas import tpu as pltpu
from jax.experimental.pallas import tpu_sc as plsc

N = 10000
E = 320000
D = 128
DH = 256
G = 64

NC = 2
NS = 16
DHALF = D // 2
NPAD = 10240
ROWS_PER_TILE = NPAD // NS
ET = E // NS
EB = 800


def _make_agg_kernel():
    mesh = plsc.VectorSubcoreMesh(core_axis_name="c", subcore_axis_name="s")

    @functools.partial(
        pl.kernel,
        mesh=mesh,
        out_type=jax.ShapeDtypeStruct((NC * NPAD, DHALF), jnp.float32),
        scratch_types=[
            pltpu.VMEM((EB,), jnp.int32),
            pltpu.VMEM((EB,), jnp.int32),
            pltpu.VMEM((EB, DHALF), jnp.float32),
            pltpu.VMEM_SHARED((NPAD, DHALF), jnp.float32),
            pltpu.SemaphoreType.DMA,
        ],
        compiler_params=pltpu.CompilerParams(use_tc_tiling_on_sc=False),
    )
    def agg(h2d_hbm, src2_hbm, dst_hbm, zeros_hbm, out_hbm,
            src_v, dst_v, rows_v, acc_sh, sem):
        cid = lax.axis_index("c")
        sid = lax.axis_index("s")

        pltpu.sync_copy(zeros_hbm.at[pl.ds(sid * ROWS_PER_TILE, ROWS_PER_TILE)],
                        acc_sh.at[pl.ds(sid * ROWS_PER_TILE, ROWS_PER_TILE)])
        plsc.subcore_barrier()

        sbase = cid * E + sid * ET
        dbase = sid * ET

        def body(j, carry):
            pltpu.sync_copy(src2_hbm.at[pl.ds(sbase + j * EB, EB)], src_v)
            pltpu.sync_copy(dst_hbm.at[pl.ds(dbase + j * EB, EB)], dst_v)
            pltpu.async_copy(h2d_hbm.at[src_v], rows_v, sem).wait()
            pltpu.sync_copy(rows_v, acc_sh.at[dst_v], add=True)
            return carry

        lax.fori_loop(0, ET // EB, body, 0)
        plsc.subcore_barrier()

        pltpu.sync_copy(
            acc_sh.at[pl.ds(sid * ROWS_PER_TILE, ROWS_PER_TILE)],
            out_hbm.at[pl.ds(cid * NPAD + sid * ROWS_PER_TILE, ROWS_PER_TILE)])

    return agg


_agg_call = _make_agg_kernel()


def _dense_body(x_ref, a0_ref, a1_ref, w1_ref, w2_ref, mg_ref, mb_ref,
                og_ref, ob_ref, b_ref, h_ref, p_ref):
    h2 = x_ref[...] + jnp.concatenate([a0_ref[...], a1_ref[...]], axis=1)
    t = jnp.dot(h2, w1_ref[...], preferred_element_type=jnp.float32)
    m = jnp.mean(t, axis=0, keepdims=True)
    d = t - m
    v = jnp.mean(d * d, axis=0, keepdims=True)
    tn = jnp.maximum(d * (mg_ref[...] * lax.rsqrt(v + 1e-5)) + mb_ref[...], 0.0)
    h3 = jnp.dot(tn, w2_ref[...], preferred_element_type=jnp.float32)
    m2 = jnp.mean(h3, axis=0, keepdims=True)
    d3 = h3 - m2
    v2 = jnp.mean(d3 * d3, axis=0, keepdims=True)
    h4 = jnp.maximum(d3 * (og_ref[...] * lax.rsqrt(v2 + 1e-5)) + ob_ref[...], 0.0)
    h_ref[...] = h4
    seg = lax.broadcasted_iota(jnp.int32, (G, N), 0)
    onehot = (seg == b_ref[...]).astype(jnp.float32)
    p_ref[...] = jnp.dot(onehot, h4, preferred_element_type=jnp.float32)


def _dense(x, a0, a1, W1, W2, mg, mb, og, ob, batch2d):
    return pl.pallas_call(
        _dense_body,
        out_shape=(jax.ShapeDtypeStruct((N, D), jnp.float32),
                   jax.ShapeDtypeStruct((G, D), jnp.float32)),
    )(x, a0, a1, W1, W2, mg.reshape(1, DH), mb.reshape(1, DH),
      og.reshape(1, D), ob.reshape(1, D), batch2d)


def kernel(x, edge_index, batch, W1_0, W2_0, mg0, mb0, og0, ob0,
           W1_1, W2_1, mg1, mb1, og1, ob1):
    src = edge_index[0].astype(jnp.int32)
    dst = edge_index[1].astype(jnp.int32)
    src2 = jnp.concatenate([2 * src, 2 * src + 1])
    zeros = jnp.zeros((NPAD, DHALF), jnp.float32)
    batch2d = batch.astype(jnp.int32).reshape(1, N)

    agg = _agg_call(x.reshape(2 * N, DHALF), src2, dst, zeros)
    h1, p0 = _dense(x, agg[:N], agg[NPAD:NPAD + N],
                    W1_0, W2_0, mg0, mb0, og0, ob0, batch2d)
    agg2 = _agg_call(h1.reshape(2 * N, DHALF), src2, dst, zeros)
    h2, p1 = _dense(h1, agg2[:N], agg2[NPAD:NPAD + N],
                    W1_1, W2_1, mg1, mb1, og1, ob1, batch2d)
    return (h2, jnp.concatenate([p0, p1], axis=1))


# speedup vs baseline: 9.2836x; 1.1967x over previous
import functools

import jax
import jax.numpy as jnp
from jax import lax
from jax.experimental import pallas as pl
from jax.experimental.pallas import tpu as pltpu
from jax.experimental.pallas import tpu_sc as plsc

N = 10000
E = 320000
D = 128
DH = 256
G = 64

NC = 2
NS = 16
DHALF = D // 2
NPAD = 10240
ROWS_PER_TILE = NPAD // NS
ET = E // NS
EB = 400
NCHUNK = ET // EB


def _make_agg_kernel():
    mesh = plsc.VectorSubcoreMesh(core_axis_name="c", subcore_axis_name="s")

    @functools.partial(
        pl.kernel,
        mesh=mesh,
        out_type=jax.ShapeDtypeStruct((NC * NPAD, DHALF), jnp.float32),
        scratch_types=[
            pltpu.VMEM((2, EB), jnp.int32),
            pltpu.VMEM((2, EB), jnp.int32),
            pltpu.VMEM((EB, DHALF), jnp.float32),
            pltpu.VMEM((EB, DHALF), jnp.float32),
            pltpu.VMEM_SHARED((NPAD, DHALF), jnp.float32),
            pltpu.SemaphoreType.DMA,
            pltpu.SemaphoreType.DMA,
        ],
        compiler_params=pltpu.CompilerParams(use_tc_tiling_on_sc=False),
    )
    def agg(h2d_hbm, idx_hbm, zeros_hbm, out_hbm,
            idx_a, idx_b, rows_a, rows_b, acc_sh, sem_a, sem_b):
        cid = lax.axis_index("c")
        sid = lax.axis_index("s")

        pltpu.sync_copy(zeros_hbm.at[pl.ds(sid * ROWS_PER_TILE, ROWS_PER_TILE)],
                        acc_sh.at[pl.ds(sid * ROWS_PER_TILE, ROWS_PER_TILE)])
        plsc.subcore_barrier()

        ibase = (cid * NS + sid) * NCHUNK * 2

        def fetch(j, idx_v, rows_v, sem):
            pltpu.sync_copy(idx_hbm.at[pl.ds(ibase + j * 2, 2)], idx_v)
            pltpu.async_copy(h2d_hbm.at[idx_v.at[0]], rows_v, sem)

        def drain(idx_v, rows_v, sem):
            pltpu.make_async_copy(h2d_hbm.at[idx_v.at[0]], rows_v, sem).wait()
            pltpu.sync_copy(rows_v, acc_sh.at[idx_v.at[1]], add=True)

        fetch(0, idx_a, rows_a, sem_a)
        fetch(1, idx_b, rows_b, sem_b)

        def body(i, carry):
            drain(idx_a, rows_a, sem_a)
            fetch(2 * i + 2, idx_a, rows_a, sem_a)
            drain(idx_b, rows_b, sem_b)
            fetch(2 * i + 3, idx_b, rows_b, sem_b)
            return carry

        lax.fori_loop(0, NCHUNK // 2 - 1, body, 0)
        drain(idx_a, rows_a, sem_a)
        drain(idx_b, rows_b, sem_b)
        plsc.subcore_barrier()

        pltpu.sync_copy(
            acc_sh.at[pl.ds(sid * ROWS_PER_TILE, ROWS_PER_TILE)],
            out_hbm.at[pl.ds(cid * NPAD + sid * ROWS_PER_TILE, ROWS_PER_TILE)])

    return agg


_agg_call = _make_agg_kernel()


def _dense_body(x_ref, a0_ref, a1_ref, w1_ref, w2_ref, mg_ref, mb_ref,
                og_ref, ob_ref, b_ref, h_ref, p_ref):
    h2 = x_ref[...] + jnp.concatenate([a0_ref[...], a1_ref[...]], axis=1)
    t = jnp.dot(h2, w1_ref[...], preferred_element_type=jnp.float32)
    m = jnp.mean(t, axis=0, keepdims=True)
    d = t - m
    v = jnp.mean(d * d, axis=0, keepdims=True)
    tn = jnp.maximum(d * (mg_ref[...] * lax.rsqrt(v + 1e-5)) + mb_ref[...], 0.0)
    h3 = jnp.dot(tn, w2_ref[...], preferred_element_type=jnp.float32)
    m2 = jnp.mean(h3, axis=0, keepdims=True)
    d3 = h3 - m2
    v2 = jnp.mean(d3 * d3, axis=0, keepdims=True)
    h4 = jnp.maximum(d3 * (og_ref[...] * lax.rsqrt(v2 + 1e-5)) + ob_ref[...], 0.0)
    h_ref[...] = h4
    seg = lax.broadcasted_iota(jnp.int32, (G, N), 0)
    onehot = (seg == b_ref[...]).astype(jnp.float32)
    p_ref[...] = jnp.dot(onehot, h4, preferred_element_type=jnp.float32)


def _dense(x, a0, a1, W1, W2, mg, mb, og, ob, batch2d):
    return pl.pallas_call(
        _dense_body,
        out_shape=(jax.ShapeDtypeStruct((N, D), jnp.float32),
                   jax.ShapeDtypeStruct((G, D), jnp.float32)),
    )(x, a0, a1, W1, W2, mg.reshape(1, DH), mb.reshape(1, DH),
      og.reshape(1, D), ob.reshape(1, D), batch2d)


def kernel(x, edge_index, batch, W1_0, W2_0, mg0, mb0, og0, ob0,
           W1_1, W2_1, mg1, mb1, og1, ob1):
    src = edge_index[0].astype(jnp.int32)
    dst = edge_index[1].astype(jnp.int32)
    s3 = jnp.stack([2 * src, 2 * src + 1]).reshape(NC, NS, NCHUNK, EB)
    d3 = jnp.broadcast_to(dst.reshape(1, NS, NCHUNK, EB), (NC, NS, NCHUNK, EB))
    idx_packed = jnp.stack([s3, d3], axis=3).reshape(NC * NS * NCHUNK * 2, EB)
    zeros = jnp.zeros((NPAD, DHALF), jnp.float32)
    batch2d = batch.astype(jnp.int32).reshape(1, N)

    agg = _agg_call(x.reshape(2 * N, DHALF), idx_packed, zeros)
    h1, p0 = _dense(x, agg[:N], agg[NPAD:NPAD + N],
                    W1_0, W2_0, mg0, mb0, og0, ob0, batch2d)
    agg2 = _agg_call(h1.reshape(2 * N, DHALF), idx_packed, zeros)
    h2, p1 = _dense(h1, agg2[:N], agg2[NPAD:NPAD + N],
                    W1_1, W2_1, mg1, mb1, og1, ob1, batch2d)
    return (h2, jnp.concatenate([p0, p1], axis=1))
